# baseline (device time: 18793 ns/iter reference)
import jax
import jax.numpy as jnp
from jax import lax
from jax.experimental import pallas as pl
from jax.experimental.pallas import tpu as pltpu

N_DEV = 4


def kernel(x):
    _, m, n = x.shape
    chunk = n // N_DEV

    def body(x_ref, out_ref, r_l, r_r, r_o, sems):
        my_x = lax.axis_index("x")
        my_y = lax.axis_index("y")
        my_z = lax.axis_index("z")
        left = lax.rem(my_z + N_DEV - 1, N_DEV)
        right = lax.rem(my_z + 1, N_DEV)
        opp = lax.rem(my_z + 2, N_DEV)

        barrier_sem = pltpu.get_barrier_semaphore()
        for nbr in (left, right, opp):
            pl.semaphore_signal(
                barrier_sem,
                inc=1,
                device_id=(my_x, my_y, nbr),
                device_id_type=pl.DeviceIdType.MESH,
            )
        pl.semaphore_wait(barrier_sem, 3)

        def push(c, dst, sem_idx, dev_z):
            return pltpu.make_async_remote_copy(
                src_ref=x_ref.at[0, :, pl.ds(c * chunk, chunk)],
                dst_ref=dst,
                send_sem=sems.at[sem_idx],
                recv_sem=sems.at[sem_idx + 3],
                device_id=(my_x, my_y, dev_z),
                device_id_type=pl.DeviceIdType.MESH,
            )

        rd_r = push(right, r_l, 0, right)
        rd_l = push(left, r_r, 1, left)
        rd_o = push(opp, r_o, 2, opp)
        rd_r.start()
        rd_l.start()
        rd_o.start()

        rd_r.wait()
        rd_l.wait()
        rd_o.wait()
        out_ref[:, :] = (
            x_ref[0, :, pl.ds(my_z * chunk, chunk)]
            + r_l[:, :]
            + r_r[:, :]
            + r_o[:, :]
        )

    return pl.pallas_call(
        body,
        out_shape=jax.ShapeDtypeStruct((m, chunk), jnp.float32),
        in_specs=[pl.BlockSpec(memory_space=pltpu.VMEM)],
        out_specs=pl.BlockSpec(memory_space=pltpu.VMEM),
        scratch_shapes=[
            pltpu.VMEM((m, chunk), jnp.float32),
            pltpu.VMEM((m, chunk), jnp.float32),
            pltpu.VMEM((m, chunk), jnp.float32),
            pltpu.SemaphoreType.DMA((6,)),
        ],
        compiler_params=pltpu.CompilerParams(collective_id=0),
    )(x)


# device time: 16073 ns/iter; 1.1692x vs baseline; 1.1692x over previous
import jax
import jax.numpy as jnp
from jax import lax
from jax.experimental import pallas as pl
from jax.experimental.pallas import tpu as pltpu

N_DEV = 4


def kernel(x):
    _, m, n = x.shape
    chunk = n // N_DEV
    half = chunk // 2

    def body(x_ref, out_ref, s_r, s_l, in_l, in_r, f_l, f_r, fwd_r, fwd_l, sems):
        my_x = lax.axis_index("x")
        my_y = lax.axis_index("y")
        my_z = lax.axis_index("z")
        left = lax.rem(my_z + N_DEV - 1, N_DEV)
        right = lax.rem(my_z + 1, N_DEV)
        opp = lax.rem(my_z + 2, N_DEV)

        def xs(c, h):
            return x_ref[0, :, pl.ds(c * chunk + h * half, half)]

        s_r[:, pl.ds(0, half)] = xs(opp, 0)
        s_r[:, pl.ds(half, half)] = xs(right, 1)
        s_l[:, pl.ds(0, half)] = xs(opp, 1)
        s_l[:, pl.ds(half, half)] = xs(left, 0)

        barrier_sem = pltpu.get_barrier_semaphore()
        for nbr in (left, right):
            pl.semaphore_signal(
                barrier_sem,
                inc=1,
                device_id=(my_x, my_y, nbr),
                device_id_type=pl.DeviceIdType.MESH,
            )
        pl.semaphore_wait(barrier_sem, 2)

        def copy(src, dst, sem_idx, dev_z):
            return pltpu.make_async_remote_copy(
                src_ref=src,
                dst_ref=dst,
                send_sem=sems.at[sem_idx],
                recv_sem=sems.at[sem_idx + 4],
                device_id=(my_x, my_y, dev_z),
                device_id_type=pl.DeviceIdType.MESH,
            )

        r1_r = copy(s_r, in_l, 0, right)
        r1_l = copy(s_l, in_r, 1, left)
        r1_r.start()
        r1_l.start()

        r1_r.wait()
        fwd_r[:, :] = xs(right, 0) + in_l[:, pl.ds(0, half)]
        r2_r = copy(fwd_r, f_l, 2, right)
        r2_r.start()

        r1_l.wait()
        fwd_l[:, :] = xs(left, 1) + in_r[:, pl.ds(0, half)]
        r2_l = copy(fwd_l, f_r, 3, left)
        r2_l.start()

        r2_r.wait()
        r2_l.wait()
        out_ref[:, pl.ds(0, half)] = (
            xs(my_z, 0) + in_r[:, pl.ds(half, half)] + f_l[:, :]
        )
        out_ref[:, pl.ds(half, half)] = (
            xs(my_z, 1) + in_l[:, pl.ds(half, half)] + f_r[:, :]
        )

    return pl.pallas_call(
        body,
        out_shape=jax.ShapeDtypeStruct((m, chunk), jnp.float32),
        in_specs=[pl.BlockSpec(memory_space=pltpu.VMEM)],
        out_specs=pl.BlockSpec(memory_space=pltpu.VMEM),
        scratch_shapes=[
            pltpu.VMEM((m, chunk), jnp.float32),
            pltpu.VMEM((m, chunk), jnp.float32),
            pltpu.VMEM((m, chunk), jnp.float32),
            pltpu.VMEM((m, chunk), jnp.float32),
            pltpu.VMEM((m, half), jnp.float32),
            pltpu.VMEM((m, half), jnp.float32),
            pltpu.VMEM((m, half), jnp.float32),
            pltpu.VMEM((m, half), jnp.float32),
            pltpu.SemaphoreType.DMA((8,)),
        ],
        compiler_params=pltpu.CompilerParams(collective_id=0),
    )(x)


# device time: 15378 ns/iter; 1.2221x vs baseline; 1.0452x over previous
import jax
import jax.numpy as jnp
from jax import lax
from jax.experimental import pallas as pl
from jax.experimental.pallas import tpu as pltpu

N_DEV = 4


def kernel(x):
    _, m, n = x.shape
    chunk = n // N_DEV
    half = chunk // 2

    def body(x_ref, out_ref, t_l, t_r, d_l, d_r, f_l, f_r, fwd_r, fwd_l, sems):
        my_x = lax.axis_index("x")
        my_y = lax.axis_index("y")
        my_z = lax.axis_index("z")
        left = lax.rem(my_z + N_DEV - 1, N_DEV)
        right = lax.rem(my_z + 1, N_DEV)
        opp = lax.rem(my_z + 2, N_DEV)

        def xs(c, h):
            return x_ref[0, :, pl.ds(c * chunk + h * half, half)]

        def xr(c, h):
            return x_ref.at[0, :, pl.ds(c * chunk + h * half, half)]

        barrier_sem = pltpu.get_barrier_semaphore()
        for nbr in (left, right):
            pl.semaphore_signal(
                barrier_sem,
                inc=1,
                device_id=(my_x, my_y, nbr),
                device_id_type=pl.DeviceIdType.MESH,
            )
        pl.semaphore_wait(barrier_sem, 2)

        def copy(src, dst, sem_idx, dev_z):
            return pltpu.make_async_remote_copy(
                src_ref=src,
                dst_ref=dst,
                send_sem=sems.at[sem_idx],
                recv_sem=sems.at[sem_idx + 6],
                device_id=(my_x, my_y, dev_z),
                device_id_type=pl.DeviceIdType.MESH,
            )

        r_t_r = copy(xr(opp, 0), t_l, 0, right)
        r_t_l = copy(xr(opp, 1), t_r, 1, left)
        r_t_r.start()
        r_t_l.start()
        r_d_r = copy(xr(right, 1), d_l, 2, right)
        r_d_l = copy(xr(left, 0), d_r, 3, left)
        r_d_r.start()
        r_d_l.start()

        r_t_r.wait()
        fwd_r[:, :] = xs(right, 0) + t_l[:, :]
        r_f_r = copy(fwd_r, f_l, 4, right)
        r_f_r.start()

        r_t_l.wait()
        fwd_l[:, :] = xs(left, 1) + t_r[:, :]
        r_f_l = copy(fwd_l, f_r, 5, left)
        r_f_l.start()

        r_d_r.wait()
        r_d_l.wait()
        r_f_r.wait()
        r_f_l.wait()
        out_ref[:, pl.ds(0, half)] = xs(my_z, 0) + d_r[:, :] + f_l[:, :]
        out_ref[:, pl.ds(half, half)] = xs(my_z, 1) + d_l[:, :] + f_r[:, :]

    return pl.pallas_call(
        body,
        out_shape=jax.ShapeDtypeStruct((m, chunk), jnp.float32),
        in_specs=[pl.BlockSpec(memory_space=pltpu.VMEM)],
        out_specs=pl.BlockSpec(memory_space=pltpu.VMEM),
        scratch_shapes=[
            pltpu.VMEM((m, half), jnp.float32),
            pltpu.VMEM((m, half), jnp.float32),
            pltpu.VMEM((m, half), jnp.float32),
            pltpu.VMEM((m, half), jnp.float32),
            pltpu.VMEM((m, half), jnp.float32),
            pltpu.VMEM((m, half), jnp.float32),
            pltpu.VMEM((m, half), jnp.float32),
            pltpu.VMEM((m, half), jnp.float32),
            pltpu.SemaphoreType.DMA((12,)),
        ],
        compiler_params=pltpu.CompilerParams(collective_id=0),
    )(x)
